# baseline (speedup 1.0000x reference)
import functools

import jax
import jax.numpy as jnp
from jax import lax
from jax.experimental import pallas as pl
from jax.experimental.pallas import tpu as pltpu
from jax.experimental.pallas import tpu_sc as plsc

NUM_EXPERTS = 8
BATCH = 4096
D_IN = 2048
D_OUT = 2048
TILE_M = 128
PADDED = BATCH + NUM_EXPERTS * TILE_M
NUM_TILES = PADDED // TILE_M

_NC, _NS = 2, 16
_NW = _NC * _NS
_CH = 16
_NBUF = 3


@functools.lru_cache(maxsize=None)
def _sc_mesh():
    return plsc.VectorSubcoreMesh(
        core_axis_name="c", subcore_axis_name="s", num_cores=_NC, num_subcores=_NS
    )


def _routing(domains):
    d = domains.astype(jnp.int32)
    onehot = (d[:, None] == jnp.arange(NUM_EXPERTS, dtype=jnp.int32)[None, :])
    oh = onehot.astype(jnp.float32)
    ohb = oh.reshape(32, 128, NUM_EXPERTS)
    tri = jnp.tril(jnp.ones((128, 128), jnp.float32))
    intra = jnp.einsum("ij,bjk->bik", tri, ohb,
                       preferred_element_type=jnp.float32)
    blocksum = jnp.sum(ohb, axis=1)
    blockpre = jnp.cumsum(blocksum, axis=0) - blocksum
    cum_incl = (intra + blockpre[:, None, :]).reshape(BATCH, NUM_EXPERTS)
    rank = jnp.sum(cum_incl * oh, axis=1).astype(jnp.int32) - 1
    counts = jnp.sum(blocksum, axis=0).astype(jnp.int32)
    padded_counts = ((counts + TILE_M - 1) // TILE_M) * TILE_M
    ends = jnp.cumsum(padded_counts)
    starts = ends - padded_counts
    dest = starts[d] + rank
    tile_ids = jnp.arange(NUM_TILES, dtype=jnp.int32) * TILE_M
    te = jnp.minimum(
        jnp.sum((ends[None, :] <= tile_ids[:, None]).astype(jnp.int32), axis=1),
        NUM_EXPERTS - 1,
    ).astype(jnp.int32)
    chg = jnp.concatenate(
        [jnp.ones((1,), jnp.int32), (te[1:] != te[:-1]).astype(jnp.int32)]
    )
    seg_idx = jnp.cumsum(chg) - 1
    wslot = (seg_idx % 2).astype(jnp.int32)
    jj = jnp.arange(NUM_TILES, dtype=jnp.int32)
    cand = jnp.where(
        (jj[None, :] > jj[:, None]) & (te[None, :] != te[:, None]),
        jj[None, :], NUM_TILES
    )
    nxt = jnp.min(cand, axis=1)
    cand2 = jnp.where(
        (jj[None, :] < jj[:, None]) & (te[None, :] != te[:, None]),
        jj[None, :], -1
    )
    prv = jnp.max(cand2, axis=1)
    prev_e = jnp.where(prv >= 0, te[jnp.maximum(prv, 0)], te)
    next_e = jnp.where(nxt < NUM_TILES,
                       te[jnp.minimum(nxt, NUM_TILES - 1)], prev_e)
    eexp = jnp.where(wslot == 0, te, next_e).astype(jnp.int32)
    oexp = jnp.where(wslot == 1, te, next_e).astype(jnp.int32)
    return dest, te, wslot, eexp, oexp


@functools.lru_cache(maxsize=None)
def _make_sc_scatter(D):
    rpw = BATCH // _NW
    nch = rpw // _CH

    @functools.partial(
        pl.kernel,
        out_type=jax.ShapeDtypeStruct((PADDED, D), jnp.float32),
        mesh=_sc_mesh(),
        scratch_types=[
            pltpu.VMEM((nch, _CH), jnp.int32),
            [pltpu.VMEM((_CH, D), jnp.float32) for _ in range(_NBUF)],
            [pltpu.SemaphoreType.DMA for _ in range(_NBUF)],
            [pltpu.SemaphoreType.DMA for _ in range(_NBUF)],
        ],
    )
    def scatter_k(rows_hbm, idx_hbm, out_hbm, idx_v, bufs, rsems, wsems):
        wid = lax.axis_index("s") * _NC + lax.axis_index("c")
        base = wid * rpw
        pltpu.sync_copy(idx_hbm.at[wid], idx_v)
        reads = [None] * nch
        writes = [None] * nch

        def start_read(c):
            reads[c] = pltpu.async_copy(
                rows_hbm.at[pl.ds(base + c * _CH, _CH)],
                bufs[c % _NBUF],
                rsems[c % _NBUF],
            )

        for c in range(min(_NBUF, nch)):
            start_read(c)
        for c in range(nch):
            reads[c].wait()
            writes[c] = pltpu.async_copy(
                bufs[c % _NBUF], out_hbm.at[idx_v.at[c]], wsems[c % _NBUF]
            )
            if c + _NBUF < nch:
                writes[c].wait()
                start_read(c + _NBUF)
        for c in range(max(0, nch - _NBUF), nch):
            writes[c].wait()

    return scatter_k


@functools.lru_cache(maxsize=None)
def _make_sc_gather(B, D):
    rpw = B // _NW
    nch = rpw // _CH

    @functools.partial(
        pl.kernel,
        out_type=jax.ShapeDtypeStruct((B, D), jnp.float32),
        mesh=_sc_mesh(),
        scratch_types=[
            pltpu.VMEM((rpw,), jnp.int32),
            [pltpu.VMEM((_CH, D), jnp.float32) for _ in range(_NBUF)],
            [pltpu.SemaphoreType.DMA for _ in range(_NBUF)],
        ],
    )
    def gather_k(table_hbm, idx_hbm, out_hbm, idx_v, bufs, sems):
        wid = lax.axis_index("s") * _NC + lax.axis_index("c")
        base = wid * rpw
        pltpu.sync_copy(idx_hbm.at[pl.ds(base, rpw)], idx_v)
        copies = [None] * nch

        def start(c):
            copies[c] = pltpu.async_copy(
                table_hbm.at[idx_v.at[pl.ds(c * _CH, _CH)]],
                bufs[c % _NBUF],
                sems[c % _NBUF],
            )

        for c in range(min(_NBUF, nch)):
            start(c)
        for c in range(nch):
            copies[c].wait()
            pltpu.sync_copy(bufs[c % _NBUF], out_hbm.at[pl.ds(base + c * _CH, _CH)])
            if c + _NBUF < nch:
                start(c + _NBUF)

    return gather_k


def _mm_body(wslot_ref, eexp_ref, oexp_ref, te_ref, x_ref, we_ref, wo_ref,
             b_ref, y_ref):
    del eexp_ref, oexp_ref, te_ref
    i = pl.program_id(1)

    @pl.when(wslot_ref[i] == 0)
    def _even():
        y_ref[...] = (
            jnp.dot(x_ref[...], we_ref[0], preferred_element_type=jnp.float32)
            + b_ref[0]
        )

    @pl.when(wslot_ref[i] == 1)
    def _odd():
        y_ref[...] = (
            jnp.dot(x_ref[...], wo_ref[0], preferred_element_type=jnp.float32)
            + b_ref[0]
        )


N_SPLIT = 2
N_CHUNK = D_OUT // N_SPLIT


def _grouped_matmul(x_sorted, W, b, te, wslot, eexp, oexp):
    grid_spec = pltpu.PrefetchScalarGridSpec(
        num_scalar_prefetch=4,
        grid=(N_SPLIT, NUM_TILES),
        in_specs=[
            pl.BlockSpec((TILE_M, D_IN), lambda n, i, ws, ee, oe, te: (i, 0)),
            pl.BlockSpec((1, D_IN, N_CHUNK),
                         lambda n, i, ws, ee, oe, te: (ee[i], 0, n)),
            pl.BlockSpec((1, D_IN, N_CHUNK),
                         lambda n, i, ws, ee, oe, te: (oe[i], 0, n)),
            pl.BlockSpec((1, 1, N_CHUNK),
                         lambda n, i, ws, ee, oe, te: (te[i], 0, n)),
        ],
        out_specs=pl.BlockSpec((TILE_M, N_CHUNK),
                               lambda n, i, ws, ee, oe, te: (i, n)),
    )
    return pl.pallas_call(
        _mm_body,
        grid_spec=grid_spec,
        out_shape=jax.ShapeDtypeStruct((PADDED, D_OUT), jnp.float32),
    )(wslot, eexp, oexp, te, x_sorted, W, W, b.reshape(NUM_EXPERTS, 1, D_OUT))


def kernel(images, domains, W, b):
    dest, te, wslot, eexp, oexp = _routing(domains)
    idx3 = dest.reshape(_NW, BATCH // _NW // _CH, _CH)
    x_sorted = _make_sc_scatter(D_IN)(images, idx3)
    y_sorted = _grouped_matmul(x_sorted, W, b, te, wslot, eexp, oexp)
    outputs = _make_sc_gather(BATCH, D_OUT)(y_sorted, dest)
    return outputs


# speedup vs baseline: 1.1719x; 1.1719x over previous
import functools

import jax
import jax.numpy as jnp
from jax import lax
from jax.experimental import pallas as pl
from jax.experimental.pallas import tpu as pltpu
from jax.experimental.pallas import tpu_sc as plsc

NUM_EXPERTS = 8
BATCH = 4096
D_IN = 2048
D_OUT = 2048
TILE_M = 128
PADDED = BATCH + NUM_EXPERTS * TILE_M
NUM_TILES = PADDED // TILE_M

_NC, _NS = 2, 16
_NW = _NC * _NS
_CH = 16
_NBUF = 3


@functools.lru_cache(maxsize=None)
def _sc_mesh():
    return plsc.VectorSubcoreMesh(
        core_axis_name="c", subcore_axis_name="s", num_cores=_NC, num_subcores=_NS
    )


def _routing(domains):
    d = domains.astype(jnp.int32)
    onehot = (d[:, None] == jnp.arange(NUM_EXPERTS, dtype=jnp.int32)[None, :])
    oh = onehot.astype(jnp.float32)
    ohb = oh.reshape(32, 128, NUM_EXPERTS)
    tri = jnp.tril(jnp.ones((128, 128), jnp.float32))
    intra = jnp.einsum("ij,bjk->bik", tri, ohb,
                       preferred_element_type=jnp.float32)
    blocksum = jnp.sum(ohb, axis=1)
    blockpre = jnp.cumsum(blocksum, axis=0) - blocksum
    cum_incl = (intra + blockpre[:, None, :]).reshape(BATCH, NUM_EXPERTS)
    rank = jnp.sum(cum_incl * oh, axis=1).astype(jnp.int32) - 1
    counts = jnp.sum(blocksum, axis=0).astype(jnp.int32)
    padded_counts = ((counts + TILE_M - 1) // TILE_M) * TILE_M
    ends = jnp.cumsum(padded_counts)
    starts = ends - padded_counts
    dest = starts[d] + rank
    tile_ids = jnp.arange(NUM_TILES, dtype=jnp.int32) * TILE_M
    tile_expert = jnp.minimum(
        jnp.sum((ends[None, :] <= tile_ids[:, None]).astype(jnp.int32), axis=1),
        NUM_EXPERTS - 1,
    ).astype(jnp.int32)
    return dest, tile_expert


@functools.lru_cache(maxsize=None)
def _make_sc_scatter(D):
    rpw = BATCH // _NW
    nch = rpw // _CH

    @functools.partial(
        pl.kernel,
        out_type=jax.ShapeDtypeStruct((PADDED, D), jnp.float32),
        mesh=_sc_mesh(),
        scratch_types=[
            pltpu.VMEM((nch, _CH), jnp.int32),
            [pltpu.VMEM((_CH, D), jnp.float32) for _ in range(_NBUF)],
            [pltpu.SemaphoreType.DMA for _ in range(_NBUF)],
            [pltpu.SemaphoreType.DMA for _ in range(_NBUF)],
        ],
    )
    def scatter_k(rows_hbm, idx_hbm, out_hbm, idx_v, bufs, rsems, wsems):
        wid = lax.axis_index("s") * _NC + lax.axis_index("c")
        base = wid * rpw
        pltpu.sync_copy(idx_hbm.at[wid], idx_v)
        reads = [None] * nch
        writes = [None] * nch

        def start_read(c):
            reads[c] = pltpu.async_copy(
                rows_hbm.at[pl.ds(base + c * _CH, _CH)],
                bufs[c % _NBUF],
                rsems[c % _NBUF],
            )

        for c in range(min(_NBUF, nch)):
            start_read(c)
        for c in range(nch):
            reads[c].wait()
            writes[c] = pltpu.async_copy(
                bufs[c % _NBUF], out_hbm.at[idx_v.at[c]], wsems[c % _NBUF]
            )
            if c + _NBUF < nch:
                writes[c].wait()
                start_read(c + _NBUF)
        for c in range(max(0, nch - _NBUF), nch):
            writes[c].wait()

    return scatter_k


@functools.lru_cache(maxsize=None)
def _make_sc_gather(B, D):
    rpw = B // _NW
    nch = rpw // _CH

    @functools.partial(
        pl.kernel,
        out_type=jax.ShapeDtypeStruct((B, D), jnp.float32),
        mesh=_sc_mesh(),
        scratch_types=[
            pltpu.VMEM((rpw,), jnp.int32),
            [pltpu.VMEM((_CH, D), jnp.float32) for _ in range(_NBUF)],
            [pltpu.SemaphoreType.DMA for _ in range(_NBUF)],
        ],
    )
    def gather_k(table_hbm, idx_hbm, out_hbm, idx_v, bufs, sems):
        wid = lax.axis_index("s") * _NC + lax.axis_index("c")
        base = wid * rpw
        pltpu.sync_copy(idx_hbm.at[pl.ds(base, rpw)], idx_v)
        copies = [None] * nch

        def start(c):
            copies[c] = pltpu.async_copy(
                table_hbm.at[idx_v.at[pl.ds(c * _CH, _CH)]],
                bufs[c % _NBUF],
                sems[c % _NBUF],
            )

        for c in range(min(_NBUF, nch)):
            start(c)
        for c in range(nch):
            copies[c].wait()
            pltpu.sync_copy(bufs[c % _NBUF], out_hbm.at[pl.ds(base + c * _CH, _CH)])
            if c + _NBUF < nch:
                start(c + _NBUF)

    return gather_k


def _mm_body(te_ref, x_ref, w_ref, b_ref, y_ref):
    del te_ref
    y_ref[...] = (
        jnp.dot(x_ref[...], w_ref[0], preferred_element_type=jnp.float32)
        + b_ref[0]
    )


def _grouped_matmul(x_sorted, W, b, tile_expert):
    grid_spec = pltpu.PrefetchScalarGridSpec(
        num_scalar_prefetch=1,
        grid=(NUM_TILES,),
        in_specs=[
            pl.BlockSpec((TILE_M, D_IN), lambda i, te: (i, 0)),
            pl.BlockSpec((1, D_IN, D_OUT), lambda i, te: (te[i], 0, 0)),
            pl.BlockSpec((1, 1, D_OUT), lambda i, te: (te[i], 0, 0)),
        ],
        out_specs=pl.BlockSpec((TILE_M, D_OUT), lambda i, te: (i, 0)),
    )
    return pl.pallas_call(
        _mm_body,
        grid_spec=grid_spec,
        out_shape=jax.ShapeDtypeStruct((PADDED, D_OUT), jnp.float32),
    )(tile_expert, x_sorted, W, b.reshape(NUM_EXPERTS, 1, D_OUT))


def kernel(images, domains, W, b):
    dest, tile_expert = _routing(domains)
    idx3 = dest.reshape(_NW, BATCH // _NW // _CH, _CH)
    x_sorted = _make_sc_scatter(D_IN)(images, idx3)
    y_sorted = _grouped_matmul(x_sorted, W, b, tile_expert)
    outputs = _make_sc_gather(BATCH, D_OUT)(y_sorted, dest)
    return outputs
